# Initial kernel scaffold; baseline (speedup 1.0000x reference)
import jax
import jax.numpy as jnp
from jax.experimental import pallas as pl


def kernel(pred, label):
    raise NotImplementedError("write your pallas kernel here")


# speedup vs baseline: 8.6736x; 8.6736x over previous
import jax
import jax.numpy as jnp
from jax.experimental import pallas as pl

_V = 100000
_C = 256
_B = 2000
_G = _V // _B


def _nll_body(pred_ref, lab_ref, out_ref):
    i = pl.program_id(0)
    x = pred_ref[...]
    m = jnp.max(x, axis=1, keepdims=True)
    e = jnp.exp(x - m)
    s = jnp.sum(e, axis=1, keepdims=True)
    lse = m + jnp.log(s)
    lab = lab_ref[0, 0, :]
    cls = jax.lax.broadcasted_iota(jnp.int32, (_B, _C), 1)
    oh = cls == lab[:, None]
    s_part = jnp.sum(jnp.where(oh, lse - x, 0.0), axis=0, keepdims=True)
    n_part = jnp.sum(oh.astype(jnp.float32), axis=0, keepdims=True)
    part = jnp.concatenate([s_part, n_part], axis=0)

    @pl.when(i == 0)
    def _init():
        out_ref[...] = part

    @pl.when(i > 0)
    def _acc():
        out_ref[...] += part


def _combine_body(sn_ref, out_ref):
    sn = sn_ref[...]
    s_c = sn[0, :]
    counts = sn[1, :]
    w = (_V - counts) * (1.0 / _V) * (counts > 0).astype(jnp.float32)
    num = jnp.sum(w * s_c)
    den = jnp.sum(w * counts)
    out_ref[...] = jnp.reshape(num / den, (1, 1))


def kernel(pred, label):
    lab3 = jnp.reshape(label, (_G, 1, _B))
    sn = pl.pallas_call(
        _nll_body,
        grid=(_G,),
        in_specs=[
            pl.BlockSpec((_B, _C), lambda i: (i, 0)),
            pl.BlockSpec((1, 1, _B), lambda i: (i, 0, 0)),
        ],
        out_specs=pl.BlockSpec((2, _C), lambda i: (0, 0)),
        out_shape=jax.ShapeDtypeStruct((2, _C), jnp.float32),
    )(pred, lab3)
    loss = pl.pallas_call(
        _combine_body,
        out_shape=jax.ShapeDtypeStruct((1, 1), jnp.float32),
    )(sn)
    return loss[0, 0]
